# baseline (speedup 1.0000x reference)
import functools

import jax
import jax.numpy as jnp
from jax import lax
from jax.experimental import pallas as pl
from jax.experimental.pallas import tpu as pltpu
from jax.experimental.pallas import tpu_sc as plsc

CH = 128
NTILES = 32
COL = 128


def _wid(c, s):
    return s * 2 + c


def _row_split(n_nodes):
    hi = (-(-n_nodes // 16) + 7) // 8 * 8
    return hi, n_nodes - 15 * hi


def _zero_rows(zsrc, acc, r0, rows):
    off = 0
    while off + CH <= rows:
        pltpu.sync_copy(zsrc.at[pl.ds(0, CH)], acc.at[pl.ds(r0 + off, CH)])
        off += CH
    if off < rows:
        pltpu.sync_copy(zsrc.at[pl.ds(0, rows - off)],
                        acc.at[pl.ds(r0 + off, rows - off)])


def _make_deg(n_nodes, cpt, nv):
    mesh = plsc.VectorSubcoreMesh(core_axis_name="c", subcore_axis_name="s")
    rpt_hi, rpt_last = _row_split(n_nodes)

    @functools.partial(
        pl.kernel,
        mesh=mesh,
        out_type=jax.ShapeDtypeStruct((2, n_nodes, 16), jnp.float32),
        scratch_types=[
            pltpu.VMEM((cpt, CH), jnp.int32),
            pltpu.VMEM((CH, 16), jnp.float32),
            pltpu.VMEM((CH, 16), jnp.float32),
            pltpu.VMEM_SHARED((n_nodes, 16), jnp.float32),
            pltpu.SemaphoreType.DMA,
            pltpu.SemaphoreType.DMA,
        ],
    )
    def deg(dst_hbm, out_hbm, dst_v, ones_v, zero_v, acc, ssem0, ssem1):
        ssem = (ssem0, ssem1)
        c = lax.axis_index("c")
        s = lax.axis_index("s")
        base = _wid(c, s) * cpt
        pltpu.sync_copy(dst_hbm.at[pl.ds(base, cpt)], dst_v)

        @pl.loop(0, CH)
        def _(r):
            ones_v[r, :] = jnp.ones((16,), jnp.float32)
            zero_v[r, :] = jnp.zeros((16,), jnp.float32)

        r0 = s * rpt_hi

        @pl.when(s < 15)
        def _():
            _zero_rows(zero_v, acc, r0, rpt_hi)

        @pl.when(s == 15)
        def _():
            _zero_rows(zero_v, acc, r0, rpt_last)

        plsc.subcore_barrier()

        @pl.loop(0, cpt // 2)
        def _(j):
            for b in range(2):
                i = j * 2 + b
                prev = jnp.maximum(i - 2, 0)

                @pl.when((i >= 2) & (base + prev < nv))
                def _():
                    pltpu.make_async_copy(
                        ones_v, acc.at[dst_v.at[prev]], ssem[b]).wait()

                @pl.when(base + i < nv)
                def _():
                    pltpu.async_copy(ones_v, acc.at[dst_v.at[i]], ssem[b],
                                     add=True)

        for tail in (cpt - 2, cpt - 1):
            @pl.when(base + tail < nv)
            def _():
                pltpu.make_async_copy(
                    ones_v, acc.at[dst_v.at[tail]], ssem[tail % 2]).wait()

        plsc.subcore_barrier()

        @pl.when(s < 15)
        def _():
            pltpu.sync_copy(acc.at[pl.ds(r0, rpt_hi)],
                            out_hbm.at[c, pl.ds(r0, rpt_hi)])

        @pl.when(s == 15)
        def _():
            pltpu.sync_copy(acc.at[pl.ds(r0, rpt_last)],
                            out_hbm.at[c, pl.ds(r0, rpt_last)])

    return deg


def _make_agg(n_nodes, cpt, nv):
    mesh = plsc.VectorSubcoreMesh(core_axis_name="c", subcore_axis_name="s")
    rpt_hi, rpt_last = _row_split(n_nodes)

    @functools.partial(
        pl.kernel,
        mesh=mesh,
        out_type=jax.ShapeDtypeStruct((2, n_nodes, COL), jnp.float32),
        scratch_types=[
            pltpu.VMEM((cpt, CH), jnp.int32),
            pltpu.VMEM((cpt, CH), jnp.int32),
            pltpu.VMEM((CH, COL), jnp.float32),
            pltpu.VMEM((CH, COL), jnp.float32),
            pltpu.VMEM_SHARED((n_nodes, COL), jnp.float32),
            pltpu.SemaphoreType.DMA,
            pltpu.SemaphoreType.DMA,
            pltpu.SemaphoreType.DMA,
            pltpu.SemaphoreType.DMA,
        ],
    )
    def agg(src_hbm, dst_hbm, zp_hbm, out_hbm,
            src_v, dst_v, rb0, rb1, acc,
            gs0, gs1, ss0, ss1):
        c = lax.axis_index("c")
        s = lax.axis_index("s")
        base = _wid(c, s) * cpt
        pltpu.sync_copy(src_hbm.at[pl.ds(base, cpt)], src_v)
        pltpu.sync_copy(dst_hbm.at[pl.ds(base, cpt)], dst_v)

        @pl.loop(0, CH)
        def _(r):
            for jj in range(COL // 16):
                rb0[r, pl.ds(jj * 16, 16)] = jnp.zeros((16,), jnp.float32)

        r0 = s * rpt_hi

        @pl.when(s < 15)
        def _():
            _zero_rows(rb0, acc, r0, rpt_hi)

        @pl.when(s == 15)
        def _():
            _zero_rows(rb0, acc, r0, rpt_last)

        plsc.subcore_barrier()

        rbufs = (rb0, rb1)
        gsems = (gs0, gs1)
        ssems = (ss0, ss1)
        pltpu.async_copy(zp_hbm.at[src_v.at[0]], rb0, gs0)

        @pl.loop(0, cpt // 2)
        def _(j):
            for b in range(2):
                i = j * 2 + b
                valid = base + i < nv
                prev = jnp.maximum(i - 1, 0)

                @pl.when(valid)
                def _():
                    pltpu.make_async_copy(
                        zp_hbm.at[src_v.at[i]], rbufs[b], gsems[b]).wait()

                @pl.when((i >= 1) & (base + prev < nv))
                def _():
                    pltpu.make_async_copy(
                        rbufs[1 - b], acc.at[dst_v.at[prev]],
                        ssems[1 - b]).wait()

                nxt = i + 1

                @pl.when((nxt < cpt) & (base + nxt < nv))
                def _():
                    pltpu.async_copy(
                        zp_hbm.at[src_v.at[nxt]], rbufs[1 - b], gsems[1 - b])

                @pl.when(valid)
                def _():
                    pltpu.async_copy(rbufs[b], acc.at[dst_v.at[i]],
                                     ssems[b], add=True)

        @pl.when(base + cpt - 1 < nv)
        def _():
            pltpu.make_async_copy(
                rbufs[(cpt - 1) % 2], acc.at[dst_v.at[cpt - 1]],
                ssems[(cpt - 1) % 2]).wait()

        plsc.subcore_barrier()

        @pl.when(s < 15)
        def _():
            pltpu.sync_copy(acc.at[pl.ds(r0, rpt_hi)],
                            out_hbm.at[c, pl.ds(r0, rpt_hi)])

        @pl.when(s == 15)
        def _():
            pltpu.sync_copy(acc.at[pl.ds(r0, rpt_last)],
                            out_hbm.at[c, pl.ds(r0, rpt_last)])

    return agg


def _scale(dp_ref):
    d = dp_ref[0, :, 0:1] + dp_ref[1, :, 0:1] + 1.0
    return lax.rsqrt(d)


def _tc_prep(x, dp, n, t):
    def body(x_ref, dp_ref, za_ref, zb_ref):
        sc = _scale(dp_ref)
        xv = x_ref[...]
        za_ref[...] = xv[:, :COL] * sc
        zb_ref[...] = xv[:, COL:] * sc

    return pl.pallas_call(
        body,
        grid=(n // t,),
        in_specs=[pl.BlockSpec((t, 2 * COL), lambda i: (i, 0)),
                  pl.BlockSpec((2, t, 16), lambda i: (0, i, 0))],
        out_specs=[pl.BlockSpec((t, COL), lambda i: (i, 0))] * 2,
        out_shape=[jax.ShapeDtypeStruct((n, COL), jnp.float32)] * 2,
    )(x, dp)


def _tc_layer1(acc_a, acc_b, z1a, z1b, dp, W1, b1, n, t):
    def body(aa, ab, za, zb, dp_ref, w, bv, o0, o1, o2, o3):
        sc = _scale(dp_ref)
        ya = sc * (aa[0] + aa[1] + za[...])
        yb = sc * (ab[0] + ab[1] + zb[...])
        y = jnp.concatenate([ya, yb], axis=1)
        h = jnp.dot(y, w[...], preferred_element_type=jnp.float32) + bv[...]
        z2 = sc * jnp.maximum(h, 0.0)
        o0[...] = z2[:, 0 * COL:1 * COL]
        o1[...] = z2[:, 1 * COL:2 * COL]
        o2[...] = z2[:, 2 * COL:3 * COL]
        o3[...] = z2[:, 3 * COL:4 * COL]

    return pl.pallas_call(
        body,
        grid=(n // t,),
        in_specs=[pl.BlockSpec((2, t, COL), lambda i: (0, i, 0)),
                  pl.BlockSpec((2, t, COL), lambda i: (0, i, 0)),
                  pl.BlockSpec((t, COL), lambda i: (i, 0)),
                  pl.BlockSpec((t, COL), lambda i: (i, 0)),
                  pl.BlockSpec((2, t, 16), lambda i: (0, i, 0)),
                  pl.BlockSpec(W1.shape, lambda i: (0, 0)),
                  pl.BlockSpec((1, W1.shape[1]), lambda i: (0, 0))],
        out_specs=[pl.BlockSpec((t, COL), lambda i: (i, 0))] * 4,
        out_shape=[jax.ShapeDtypeStruct((n, COL), jnp.float32)] * 4,
    )(acc_a, acc_b, z1a, z1b, dp, W1, b1)


def _tc_layer2(accs, zs, dp, W2, b2, W3, n, t):
    def body(a0, a1, a2, a3, z0, z1, z2, z3, dp_ref, w2, bv, w3, o0, o1):
        sc = _scale(dp_ref)
        y = jnp.concatenate(
            [sc * (a[0] + a[1] + z[...])
             for a, z in zip((a0, a1, a2, a3), (z0, z1, z2, z3))], axis=1)
        h = jnp.maximum(
            jnp.dot(y, w2[...], preferred_element_type=jnp.float32) + bv[...], 0.0)
        tt = jnp.dot(h, w3[...], preferred_element_type=jnp.float32)
        z3p = sc * tt
        o0[...] = z3p[:, :COL]
        o1[...] = z3p[:, COL:]

    return pl.pallas_call(
        body,
        grid=(n // t,),
        in_specs=[pl.BlockSpec((2, t, COL), lambda i: (0, i, 0))] * 4
        + [pl.BlockSpec((t, COL), lambda i: (i, 0))] * 4
        + [pl.BlockSpec((2, t, 16), lambda i: (0, i, 0)),
           pl.BlockSpec(W2.shape, lambda i: (0, 0)),
           pl.BlockSpec((1, W2.shape[1]), lambda i: (0, 0)),
           pl.BlockSpec(W3.shape, lambda i: (0, 0))],
        out_specs=[pl.BlockSpec((t, COL), lambda i: (i, 0))] * 2,
        out_shape=[jax.ShapeDtypeStruct((n, COL), jnp.float32)] * 2,
    )(*accs, *zs, dp, W2, b2, W3)


def _tc_layer3(acc_a, acc_b, z0, z1, dp, b3, n, t):
    def body(aa, ab, za, zb, dp_ref, bv, o):
        sc = _scale(dp_ref)
        y = jnp.concatenate([sc * (aa[0] + aa[1] + za[...]),
                             sc * (ab[0] + ab[1] + zb[...])], axis=1) + bv[...]
        m = jnp.max(y, axis=1, keepdims=True)
        e = jnp.exp(y - m)
        o[...] = e / jnp.sum(e, axis=1, keepdims=True)

    return pl.pallas_call(
        body,
        grid=(n // t,),
        in_specs=[pl.BlockSpec((2, t, COL), lambda i: (0, i, 0)),
                  pl.BlockSpec((2, t, COL), lambda i: (0, i, 0)),
                  pl.BlockSpec((t, COL), lambda i: (i, 0)),
                  pl.BlockSpec((t, COL), lambda i: (i, 0)),
                  pl.BlockSpec((2, t, 16), lambda i: (0, i, 0)),
                  pl.BlockSpec((1, 2 * COL), lambda i: (0, 0))],
        out_specs=pl.BlockSpec((t, 2 * COL), lambda i: (i, 0)),
        out_shape=jax.ShapeDtypeStruct((n, 2 * COL), jnp.float32),
    )(acc_a, acc_b, z0, z1, dp, b3)


def kernel(x, edge_index, W1, b1, W2, b2, W3, b3):
    n = x.shape[0]
    e = edge_index.shape[1]
    t = 1000

    src = edge_index[0].astype(jnp.int32)
    dst = edge_index[1].astype(jnp.int32)
    nv = e // CH
    cpt = -(-nv // NTILES)
    npad = cpt * NTILES
    pad = npad * CH - e
    src2 = jnp.pad(src, (0, pad)).reshape(npad, CH)
    dst2 = jnp.pad(dst, (0, pad)).reshape(npad, CH)

    deg_k = _make_deg(n, cpt, nv)
    agg_k = _make_agg(n, cpt, nv)

    dp = deg_k(dst2)
    z1a, z1b = _tc_prep(x, dp, n, t)
    acc_a = agg_k(src2, dst2, z1a)
    acc_b = agg_k(src2, dst2, z1b)
    z2 = _tc_layer1(acc_a, acc_b, z1a, z1b, dp, W1, b1.reshape(1, -1), n, t)
    accs2 = [agg_k(src2, dst2, z) for z in z2]
    z3a, z3b = _tc_layer2(accs2, z2, dp, W2, b2.reshape(1, -1), W3, n, t)
    acc3a = agg_k(src2, dst2, z3a)
    acc3b = agg_k(src2, dst2, z3b)
    return _tc_layer3(acc3a, acc3b, z3a, z3b, dp, b3.reshape(1, -1), n, t)


# speedup vs baseline: 1.0170x; 1.0170x over previous
import functools

import jax
import jax.numpy as jnp
from jax import lax
from jax.experimental import pallas as pl
from jax.experimental.pallas import tpu as pltpu
from jax.experimental.pallas import tpu_sc as plsc

CH = 128
NTILES = 32
COL = 128
K = 1
ZR = 32


def _wid(c, s):
    return s * 2 + c


def _row_split(n_nodes):
    hi = (-(-n_nodes // 16) + 7) // 8 * 8
    return hi, n_nodes - 15 * hi


def _zero_rows(zsrc, acc, r0, rows):
    off = 0
    while off + ZR <= rows:
        pltpu.sync_copy(zsrc.at[pl.ds(0, ZR)], acc.at[pl.ds(r0 + off, ZR)])
        off += ZR
    if off < rows:
        pltpu.sync_copy(zsrc.at[pl.ds(0, rows - off)],
                        acc.at[pl.ds(r0 + off, rows - off)])


def _per_tile_span(s, rpt_hi, rpt_last, fn):
    @pl.when(s < 15)
    def _():
        fn(rpt_hi)

    @pl.when(s == 15)
    def _():
        fn(rpt_last)


def _make_deg(n_nodes, cpt, nv):
    mesh = plsc.VectorSubcoreMesh(core_axis_name="c", subcore_axis_name="s")
    rpt_hi, rpt_last = _row_split(n_nodes)

    @functools.partial(
        pl.kernel,
        mesh=mesh,
        out_type=jax.ShapeDtypeStruct((2, n_nodes, 16), jnp.float32),
        scratch_types=[
            pltpu.VMEM((cpt, CH), jnp.int32),
            pltpu.VMEM((CH, 16), jnp.float32),
            pltpu.VMEM((CH, 16), jnp.float32),
            pltpu.VMEM_SHARED((n_nodes, 16), jnp.float32),
            pltpu.SemaphoreType.DMA,
        ],
    )
    def deg(dst_hbm, out_hbm, dst_v, ones_v, zero_v, acc, ssem):
        c = lax.axis_index("c")
        s = lax.axis_index("s")
        base = _wid(c, s) * cpt
        pltpu.sync_copy(dst_hbm.at[pl.ds(base, cpt)], dst_v)

        @pl.loop(0, CH)
        def _(r):
            ones_v[r, :] = jnp.ones((16,), jnp.float32)
            zero_v[r, :] = jnp.zeros((16,), jnp.float32)

        r0 = s * rpt_hi
        _per_tile_span(s, rpt_hi, rpt_last,
                       lambda rows: _zero_rows(zero_v, acc, r0, rows))
        plsc.subcore_barrier()

        @pl.loop(0, cpt)
        def _(i):
            @pl.when(base + i < nv)
            def _():
                pltpu.async_copy(ones_v, acc.at[dst_v.at[i]], ssem, add=True)

        @pl.loop(0, cpt)
        def _(i):
            @pl.when(base + i < nv)
            def _():
                pltpu.make_async_copy(
                    ones_v, acc.at[dst_v.at[i]], ssem).wait()

        plsc.subcore_barrier()
        _per_tile_span(
            s, rpt_hi, rpt_last,
            lambda rows: pltpu.sync_copy(acc.at[pl.ds(r0, rows)],
                                         out_hbm.at[c, pl.ds(r0, rows)]))

    return deg


def _make_agg(n_nodes, cpt, nv, nblk):
    mesh = plsc.VectorSubcoreMesh(core_axis_name="c", subcore_axis_name="s")
    rpt_hi, rpt_last = _row_split(n_nodes)
    nhalf = 2 if K > 1 else 1
    hcpt = cpt // nhalf
    nstep = hcpt // K

    @functools.partial(
        pl.kernel,
        mesh=mesh,
        out_type=[jax.ShapeDtypeStruct((2, n_nodes, COL), jnp.float32)] * nblk,
        scratch_types=[
            pltpu.VMEM((hcpt, CH), jnp.int32),
            pltpu.VMEM((hcpt, CH), jnp.int32),
        ]
        + [pltpu.VMEM((CH, COL), jnp.float32)] * (2 * K)
        + [pltpu.VMEM_SHARED((n_nodes, COL), jnp.float32),
           pltpu.SemaphoreType.DMA,
           pltpu.SemaphoreType.DMA,
           pltpu.SemaphoreType.DMA,
           pltpu.SemaphoreType.DMA],
    )
    def agg(*refs):
        src_hbm, dst_hbm = refs[0], refs[1]
        zps = refs[2:2 + nblk]
        outs = refs[2 + nblk:2 + 2 * nblk]
        sc_refs = refs[2 + 2 * nblk:]
        src_v, dst_v = sc_refs[0], sc_refs[1]
        bufs = sc_refs[2:2 + 2 * K]
        acc = sc_refs[2 + 2 * K]
        gsems = sc_refs[3 + 2 * K:5 + 2 * K]
        ssems = sc_refs[5 + 2 * K:7 + 2 * K]
        zbuf = bufs[0]

        c = lax.axis_index("c")
        s = lax.axis_index("s")
        base = _wid(c, s) * cpt

        r0 = s * rpt_hi

        def fire_gathers(zp, gbase, step, g):
            for k in range(K):
                i = step * K + k

                @pl.when((i < hcpt) & (gbase + i < nv))
                def _():
                    pltpu.async_copy(zp.at[src_v.at[i]], bufs[g * K + k],
                                     gsems[g])

        def wait_gathers(zp, gbase, step, g):
            for k in range(K):
                i = step * K + k

                @pl.when((i < hcpt) & (gbase + i < nv))
                def _():
                    pltpu.make_async_copy(
                        zp.at[src_v.at[i]], bufs[g * K + k], gsems[g]).wait()

        def fire_scatters(gbase, step, g):
            for k in range(K):
                i = step * K + k

                @pl.when((i < hcpt) & (gbase + i < nv))
                def _():
                    pltpu.async_copy(bufs[g * K + k], acc.at[dst_v.at[i]],
                                     ssems[g], add=True)

        def wait_scatters(gbase, step, g):
            for k in range(K):
                i = step * K + k
                iv = jnp.maximum(i, 0)

                @pl.when((i >= 0) & (i < hcpt) & (gbase + iv < nv))
                def _():
                    pltpu.make_async_copy(
                        bufs[g * K + k], acc.at[dst_v.at[iv]],
                        ssems[g]).wait()

        for blk in range(nblk):
            zp = zps[blk]

            @pl.loop(0, ZR)
            def _(r):
                for jj in range(COL // 16):
                    zbuf[r, pl.ds(jj * 16, 16)] = jnp.zeros(
                        (16,), jnp.float32)

            _per_tile_span(s, rpt_hi, rpt_last,
                           lambda rows: _zero_rows(zbuf, acc, r0, rows))
            plsc.subcore_barrier()

            for h in range(nhalf):
                gbase = base + h * hcpt
                pltpu.sync_copy(src_hbm.at[pl.ds(gbase, hcpt)], src_v)
                pltpu.sync_copy(dst_hbm.at[pl.ds(gbase, hcpt)], dst_v)
                fire_gathers(zp, gbase, 0, 0)

                @pl.loop(0, nstep // 2)
                def _(j, gbase=gbase):
                    for p in range(2):
                        st = j * 2 + p
                        wait_gathers(zp, gbase, st, p)
                        fire_scatters(gbase, st, p)
                        wait_scatters(gbase, st - 1, 1 - p)
                        fire_gathers(zp, gbase, st + 1, 1 - p)

                wait_scatters(gbase, nstep - 1, (nstep - 1) % 2)

            plsc.subcore_barrier()
            _per_tile_span(
                s, rpt_hi, rpt_last,
                lambda rows, blk=blk: pltpu.sync_copy(
                    acc.at[pl.ds(r0, rows)],
                    outs[blk].at[c, pl.ds(r0, rows)]))

    return agg


def _scale(dp_ref):
    d = dp_ref[0, :, 0:1] + dp_ref[1, :, 0:1] + 1.0
    return lax.rsqrt(d)


def _acc_spec(t):
    return pl.BlockSpec((2, t, COL), lambda i: (0, i, 0))


def _z_spec(t):
    return pl.BlockSpec((t, COL), lambda i: (i, 0))


def _dp_spec(t):
    return pl.BlockSpec((2, t, 16), lambda i: (0, i, 0))


def _split_out(z, refs):
    for k, o in enumerate(refs):
        o[...] = z[:, k * COL:(k + 1) * COL]


def _tc_prep(x, dp, n, t):
    def body(x_ref, dp_ref, *os):
        sc = _scale(dp_ref)
        _split_out(x_ref[...] * sc, os)

    return pl.pallas_call(
        body,
        grid=(n // t,),
        in_specs=[pl.BlockSpec((t, 256), lambda i: (i, 0)), _dp_spec(t)],
        out_specs=[_z_spec(t)] * 2,
        out_shape=[jax.ShapeDtypeStruct((n, COL), jnp.float32)] * 2,
    )(x, dp)


def _tc_layer1(accs, zs, dp, W1, b1, n, t):
    def body(*refs):
        a = refs[0:2]
        z = refs[2:4]
        dp_ref, w, bv = refs[4], refs[5], refs[6]
        os = refs[7:]
        sc = _scale(dp_ref)
        y = jnp.concatenate(
            [sc * (ak[0] + ak[1] + zk[...]) for ak, zk in zip(a, z)], axis=1)
        h = jnp.dot(y, w[...], preferred_element_type=jnp.float32) + bv[...]
        _split_out(sc * jnp.maximum(h, 0.0), os)

    return pl.pallas_call(
        body,
        grid=(n // t,),
        in_specs=[_acc_spec(t)] * 2 + [_z_spec(t)] * 2
        + [_dp_spec(t),
           pl.BlockSpec(W1.shape, lambda i: (0, 0)),
           pl.BlockSpec((1, W1.shape[1]), lambda i: (0, 0))],
        out_specs=[_z_spec(t)] * 4,
        out_shape=[jax.ShapeDtypeStruct((n, COL), jnp.float32)] * 4,
    )(*accs, *zs, dp, W1, b1)


def _tc_layer2(accs, zs, dp, W2, b2, W3, n, t):
    def body(*refs):
        a = refs[0:4]
        z = refs[4:8]
        dp_ref, w2, bv, w3 = refs[8], refs[9], refs[10], refs[11]
        os = refs[12:]
        sc = _scale(dp_ref)
        y = jnp.concatenate(
            [sc * (ak[0] + ak[1] + zk[...]) for ak, zk in zip(a, z)], axis=1)
        h = jnp.maximum(
            jnp.dot(y, w2[...], preferred_element_type=jnp.float32) + bv[...],
            0.0)
        tt = jnp.dot(h, w3[...], preferred_element_type=jnp.float32)
        _split_out(sc * tt, os)

    return pl.pallas_call(
        body,
        grid=(n // t,),
        in_specs=[_acc_spec(t)] * 4 + [_z_spec(t)] * 4
        + [_dp_spec(t),
           pl.BlockSpec(W2.shape, lambda i: (0, 0)),
           pl.BlockSpec((1, W2.shape[1]), lambda i: (0, 0)),
           pl.BlockSpec(W3.shape, lambda i: (0, 0))],
        out_specs=[_z_spec(t)] * 2,
        out_shape=[jax.ShapeDtypeStruct((n, COL), jnp.float32)] * 2,
    )(*accs, *zs, dp, W2, b2, W3)


def _tc_layer3(accs, zs, dp, b3, n, t):
    def body(*refs):
        a = refs[0:2]
        z = refs[2:4]
        dp_ref, bv, o = refs[4], refs[5], refs[6]
        sc = _scale(dp_ref)
        y = jnp.concatenate(
            [sc * (ak[0] + ak[1] + zk[...]) for ak, zk in zip(a, z)],
            axis=1) + bv[...]
        m = jnp.max(y, axis=1, keepdims=True)
        e = jnp.exp(y - m)
        o[...] = e / jnp.sum(e, axis=1, keepdims=True)

    return pl.pallas_call(
        body,
        grid=(n // t,),
        in_specs=[_acc_spec(t)] * 2 + [_z_spec(t)] * 2
        + [_dp_spec(t), pl.BlockSpec((1, 256), lambda i: (0, 0))],
        out_specs=pl.BlockSpec((t, 256), lambda i: (i, 0)),
        out_shape=jax.ShapeDtypeStruct((n, 256), jnp.float32),
    )(*accs, *zs, dp, b3)


def kernel(x, edge_index, W1, b1, W2, b2, W3, b3):
    n = x.shape[0]
    e = edge_index.shape[1]
    t = 1000

    src = edge_index[0].astype(jnp.int32)
    dst = edge_index[1].astype(jnp.int32)
    nv = e // CH
    cpt = -(-nv // NTILES)
    cpt = -(-cpt // (2 * K)) * (2 * K)
    npad = cpt * NTILES
    pad = npad * CH - e
    src2 = jnp.pad(src, (0, pad)).reshape(npad, CH)
    dst2 = jnp.pad(dst, (0, pad)).reshape(npad, CH)

    deg_k = _make_deg(n, cpt, nv)
    agg2 = _make_agg(n, cpt, nv, 2)
    agg4 = _make_agg(n, cpt, nv, 4)

    dp = deg_k(dst2)
    z1 = _tc_prep(x, dp, n, t)
    acc1 = agg2(src2, dst2, *z1)
    z2 = _tc_layer1(acc1, z1, dp, W1, b1.reshape(1, -1), n, t)
    acc2 = agg4(src2, dst2, *z2)
    z3 = _tc_layer2(acc2, z2, dp, W2, b2.reshape(1, -1), W3, n, t)
    acc3 = agg2(src2, dst2, *z3)
    return _tc_layer3(acc3, z3, dp, b3.reshape(1, -1), n, t)


# speedup vs baseline: 1.0313x; 1.0141x over previous
import functools

import jax
import jax.numpy as jnp
from jax import lax
from jax.experimental import pallas as pl
from jax.experimental.pallas import tpu as pltpu
from jax.experimental.pallas import tpu_sc as plsc

CH = 128
NTILES = 32
COL = 128
ZR = 32


def _wid(c, s):
    return s * 2 + c


def _row_split(n_nodes):
    hi = (-(-n_nodes // 16) + 7) // 8 * 8
    return hi, n_nodes - 15 * hi


def _zero_rows(zsrc, acc, r0, rows):
    off = 0
    while off + ZR <= rows:
        pltpu.sync_copy(zsrc.at[pl.ds(0, ZR)], acc.at[pl.ds(r0 + off, ZR)])
        off += ZR
    if off < rows:
        pltpu.sync_copy(zsrc.at[pl.ds(0, rows - off)],
                        acc.at[pl.ds(r0 + off, rows - off)])


def _per_tile_span(s, rpt_hi, rpt_last, fn):
    @pl.when(s < 15)
    def _():
        fn(rpt_hi)

    @pl.when(s == 15)
    def _():
        fn(rpt_last)


def _make_deg(n_nodes, cpt, nv):
    mesh = plsc.VectorSubcoreMesh(core_axis_name="c", subcore_axis_name="s")
    rpt_hi, rpt_last = _row_split(n_nodes)

    @functools.partial(
        pl.kernel,
        mesh=mesh,
        out_type=jax.ShapeDtypeStruct((2, n_nodes, 16), jnp.float32),
        scratch_types=[
            pltpu.VMEM((cpt, CH), jnp.int32),
            pltpu.VMEM((CH, 16), jnp.float32),
            pltpu.VMEM((CH, 16), jnp.float32),
            pltpu.VMEM_SHARED((n_nodes, 16), jnp.float32),
            pltpu.SemaphoreType.DMA,
        ],
    )
    def deg(dst_hbm, out_hbm, dst_v, ones_v, zero_v, acc, ssem):
        c = lax.axis_index("c")
        s = lax.axis_index("s")
        base = _wid(c, s) * cpt
        pltpu.sync_copy(dst_hbm.at[pl.ds(base, cpt)], dst_v)

        @pl.loop(0, CH)
        def _(r):
            ones_v[r, :] = jnp.ones((16,), jnp.float32)
            zero_v[r, :] = jnp.zeros((16,), jnp.float32)

        r0 = s * rpt_hi
        _per_tile_span(s, rpt_hi, rpt_last,
                       lambda rows: _zero_rows(zero_v, acc, r0, rows))
        plsc.subcore_barrier()

        @pl.loop(0, cpt)
        def _(i):
            @pl.when(base + i < nv)
            def _():
                pltpu.async_copy(ones_v, acc.at[dst_v.at[i]], ssem, add=True)

        @pl.loop(0, cpt)
        def _(i):
            @pl.when(base + i < nv)
            def _():
                pltpu.make_async_copy(
                    ones_v, acc.at[dst_v.at[i]], ssem).wait()

        plsc.subcore_barrier()
        _per_tile_span(
            s, rpt_hi, rpt_last,
            lambda rows: pltpu.sync_copy(acc.at[pl.ds(r0, rows)],
                                         out_hbm.at[c, pl.ds(r0, rows)]))

    return deg


def _make_agg(n_nodes, cpt, nv, nblk):
    mesh = plsc.VectorSubcoreMesh(core_axis_name="c", subcore_axis_name="s")
    rpt_hi, rpt_last = _row_split(n_nodes)

    @functools.partial(
        pl.kernel,
        mesh=mesh,
        out_type=[jax.ShapeDtypeStruct((2, n_nodes, COL), jnp.float32)] * nblk,
        scratch_types=[
            pltpu.VMEM((cpt, CH), jnp.int32),
            pltpu.VMEM((cpt, CH), jnp.int32),
            pltpu.VMEM((CH, COL), jnp.float32),
            pltpu.VMEM((CH, COL), jnp.float32),
            pltpu.VMEM_SHARED((n_nodes, COL), jnp.float32),
            pltpu.SemaphoreType.DMA,
            pltpu.SemaphoreType.DMA,
            pltpu.SemaphoreType.DMA,
            pltpu.SemaphoreType.DMA,
        ],
    )
    def agg(*refs):
        src_hbm, dst_hbm = refs[0], refs[1]
        zps = refs[2:2 + nblk]
        outs = refs[2 + nblk:2 + 2 * nblk]
        sc_refs = refs[2 + 2 * nblk:]
        src_v, dst_v, rb0, rb1, acc = sc_refs[0:5]
        gsems = sc_refs[5:7]
        ssems = sc_refs[7:9]
        rbufs = (rb0, rb1)
        zbuf = rb0

        c = lax.axis_index("c")
        s = lax.axis_index("s")
        base = _wid(c, s) * cpt
        pltpu.sync_copy(src_hbm.at[pl.ds(base, cpt)], src_v)
        pltpu.sync_copy(dst_hbm.at[pl.ds(base, cpt)], dst_v)

        r0 = s * rpt_hi

        for blk in range(nblk):
            zp = zps[blk]

            @pl.loop(0, ZR)
            def _(r):
                for jj in range(COL // 16):
                    zbuf[r, pl.ds(jj * 16, 16)] = jnp.zeros(
                        (16,), jnp.float32)

            _per_tile_span(s, rpt_hi, rpt_last,
                           lambda rows: _zero_rows(zbuf, acc, r0, rows))
            plsc.subcore_barrier()

            pltpu.async_copy(zp.at[src_v.at[0]], rb0, gsems[0])

            @pl.loop(0, cpt // 2)
            def _(j):
                for b in range(2):
                    i = j * 2 + b
                    valid = base + i < nv
                    prev = jnp.maximum(i - 1, 0)

                    @pl.when(valid)
                    def _():
                        pltpu.make_async_copy(
                            zp.at[src_v.at[i]], rbufs[b], gsems[b]).wait()

                    @pl.when((i >= 1) & (base + prev < nv))
                    def _():
                        pltpu.make_async_copy(
                            rbufs[1 - b], acc.at[dst_v.at[prev]],
                            ssems[1 - b]).wait()

                    nxt = i + 1

                    @pl.when((nxt < cpt) & (base + nxt < nv))
                    def _():
                        pltpu.async_copy(zp.at[src_v.at[nxt]], rbufs[1 - b],
                                         gsems[1 - b])

                    @pl.when(valid)
                    def _():
                        pltpu.async_copy(rbufs[b], acc.at[dst_v.at[i]],
                                         ssems[b], add=True)

            @pl.when(base + cpt - 1 < nv)
            def _():
                pltpu.make_async_copy(
                    rbufs[(cpt - 1) % 2], acc.at[dst_v.at[cpt - 1]],
                    ssems[(cpt - 1) % 2]).wait()

            plsc.subcore_barrier()
            _per_tile_span(
                s, rpt_hi, rpt_last,
                lambda rows, blk=blk: pltpu.sync_copy(
                    acc.at[pl.ds(r0, rows)],
                    outs[blk].at[c, pl.ds(r0, rows)]))

    return agg


def _scale(dp_ref):
    d = dp_ref[0, :, 0:1] + dp_ref[1, :, 0:1] + 1.0
    return lax.rsqrt(d)


def _acc_spec(t):
    return pl.BlockSpec((2, t, COL), lambda i: (0, i, 0))


def _z_spec(t):
    return pl.BlockSpec((t, COL), lambda i: (i, 0))


def _dp_spec(t):
    return pl.BlockSpec((2, t, 16), lambda i: (0, i, 0))


def _split_out(z, refs):
    for k, o in enumerate(refs):
        o[...] = z[:, k * COL:(k + 1) * COL]


def _tc_prep(x, dp, n, t):
    def body(x_ref, dp_ref, *os):
        sc = _scale(dp_ref)
        _split_out(x_ref[...] * sc, os)

    return pl.pallas_call(
        body,
        grid=(n // t,),
        in_specs=[pl.BlockSpec((t, 256), lambda i: (i, 0)), _dp_spec(t)],
        out_specs=[_z_spec(t)] * 2,
        out_shape=[jax.ShapeDtypeStruct((n, COL), jnp.float32)] * 2,
    )(x, dp)


def _tc_layer1(accs, zs, dp, W1, b1, n, t):
    def body(*refs):
        a = refs[0:2]
        z = refs[2:4]
        dp_ref, w, bv = refs[4], refs[5], refs[6]
        os = refs[7:]
        sc = _scale(dp_ref)
        y = jnp.concatenate(
            [sc * (ak[0] + ak[1] + zk[...]) for ak, zk in zip(a, z)], axis=1)
        h = jnp.dot(y, w[...], preferred_element_type=jnp.float32) + bv[...]
        _split_out(sc * jnp.maximum(h, 0.0), os)

    return pl.pallas_call(
        body,
        grid=(n // t,),
        in_specs=[_acc_spec(t)] * 2 + [_z_spec(t)] * 2
        + [_dp_spec(t),
           pl.BlockSpec(W1.shape, lambda i: (0, 0)),
           pl.BlockSpec((1, W1.shape[1]), lambda i: (0, 0))],
        out_specs=[_z_spec(t)] * 4,
        out_shape=[jax.ShapeDtypeStruct((n, COL), jnp.float32)] * 4,
    )(*accs, *zs, dp, W1, b1)


def _tc_layer2(accs, zs, dp, W2, b2, W3, n, t):
    def body(*refs):
        a = refs[0:4]
        z = refs[4:8]
        dp_ref, w2, bv, w3 = refs[8], refs[9], refs[10], refs[11]
        os = refs[12:]
        sc = _scale(dp_ref)
        y = jnp.concatenate(
            [sc * (ak[0] + ak[1] + zk[...]) for ak, zk in zip(a, z)], axis=1)
        h = jnp.maximum(
            jnp.dot(y, w2[...], preferred_element_type=jnp.float32) + bv[...],
            0.0)
        tt = jnp.dot(h, w3[...], preferred_element_type=jnp.float32)
        _split_out(sc * tt, os)

    return pl.pallas_call(
        body,
        grid=(n // t,),
        in_specs=[_acc_spec(t)] * 4 + [_z_spec(t)] * 4
        + [_dp_spec(t),
           pl.BlockSpec(W2.shape, lambda i: (0, 0)),
           pl.BlockSpec((1, W2.shape[1]), lambda i: (0, 0)),
           pl.BlockSpec(W3.shape, lambda i: (0, 0))],
        out_specs=[_z_spec(t)] * 2,
        out_shape=[jax.ShapeDtypeStruct((n, COL), jnp.float32)] * 2,
    )(*accs, *zs, dp, W2, b2, W3)


def _tc_layer3(accs, zs, dp, b3, n, t):
    def body(*refs):
        a = refs[0:2]
        z = refs[2:4]
        dp_ref, bv, o = refs[4], refs[5], refs[6]
        sc = _scale(dp_ref)
        y = jnp.concatenate(
            [sc * (ak[0] + ak[1] + zk[...]) for ak, zk in zip(a, z)],
            axis=1) + bv[...]
        m = jnp.max(y, axis=1, keepdims=True)
        e = jnp.exp(y - m)
        o[...] = e / jnp.sum(e, axis=1, keepdims=True)

    return pl.pallas_call(
        body,
        grid=(n // t,),
        in_specs=[_acc_spec(t)] * 2 + [_z_spec(t)] * 2
        + [_dp_spec(t), pl.BlockSpec((1, 256), lambda i: (0, 0))],
        out_specs=pl.BlockSpec((t, 256), lambda i: (i, 0)),
        out_shape=jax.ShapeDtypeStruct((n, 256), jnp.float32),
    )(*accs, *zs, dp, b3)


def kernel(x, edge_index, W1, b1, W2, b2, W3, b3):
    n = x.shape[0]
    e = edge_index.shape[1]
    t = 1000

    src = edge_index[0].astype(jnp.int32)
    dst = edge_index[1].astype(jnp.int32)
    nv = e // CH
    cpt = -(-nv // NTILES)
    cpt = -(-cpt // 2) * 2
    npad = cpt * NTILES
    pad = npad * CH - e
    src2 = jnp.pad(src, (0, pad)).reshape(npad, CH)
    dst2 = jnp.pad(dst, (0, pad)).reshape(npad, CH)

    deg_k = _make_deg(n, cpt, nv)
    agg2 = _make_agg(n, cpt, nv, 2)
    agg4 = _make_agg(n, cpt, nv, 4)

    dp = deg_k(dst2)
    z1 = _tc_prep(x, dp, n, t)
    acc1 = agg2(src2, dst2, *z1)
    z2 = _tc_layer1(acc1, z1, dp, W1, b1.reshape(1, -1), n, t)
    acc2 = agg4(src2, dst2, *z2)
    z3 = _tc_layer2(acc2, z2, dp, W2, b2.reshape(1, -1), W3, n, t)
    acc3 = agg2(src2, dst2, *z3)
    return _tc_layer3(acc3, z3, dp, b3.reshape(1, -1), n, t)
